# baseline (speedup 1.0000x reference)
import functools

import jax
import jax.numpy as jnp
from jax import lax
from jax.experimental import pallas as pl
from jax.experimental.pallas import tpu as pltpu
from jax.experimental.pallas import tpu_sc as plsc

_L = 16


def _shuf(x, idx):
    dn = lax.GatherDimensionNumbers(
        offset_dims=(), collapsed_slice_dims=(0,), start_index_map=(0,))
    return lax.gather(x, idx[:, None], dn, slice_sizes=(1,),
                      mode=lax.GatherScatterMode.PROMISE_IN_BOUNDS)


def _bmax(x, lane):
    for k in (1, 2, 4, 8):
        x = jnp.maximum(x, _shuf(x, lane ^ k))
    return x


def _bmin(x, lane):
    for k in (1, 2, 4, 8):
        x = jnp.minimum(x, _shuf(x, lane ^ k))
    return x


def _bsum(x, lane):
    for k in (1, 2, 4, 8):
        x = x + _shuf(x, lane ^ k)
    return x


def _prefix(x, lane):
    for k in (1, 2, 4, 8):
        sh = _shuf(x, jnp.maximum(lane - k, 0))
        x = x + jnp.where(lane >= k, sh, 0.0)
    return x


def _sc_router_body(logits_ref, ct_ref, meta_ref, cnt_hbm, row_v, ct_v,
                    cnt_loc, cnt_all, meta_v):
    c = lax.axis_index("c")
    s = lax.axis_index("s")
    t = ct_ref.shape[0]
    n_e = ct_ref.shape[1]
    n_ch = n_e // _L
    tok_per = t // 16
    lane = lax.broadcasted_iota(jnp.int32, (_L,), 0)

    @pl.when(c == 0)
    def _route():
        base = s * tok_per
        pltpu.sync_copy(logits_ref.at[pl.ds(base, tok_per)], row_v)
        accs = [jnp.zeros((_L,), jnp.float32) for _ in range(n_ch)]
        for tk in range(tok_per):
            chunks = [row_v[tk, pl.ds(ch * _L, _L)] for ch in range(n_ch)]
            m = chunks[0]
            for ch in range(1, n_ch):
                m = jnp.maximum(m, chunks[ch])
            m = _bmax(m, lane)
            exs = [jnp.exp(ck - m) for ck in chunks]
            den0 = exs[0]
            for ch in range(1, n_ch):
                den0 = den0 + exs[ch]
            den_t = _bsum(den0, lane)
            ecs = [ex / den_t for ex in exs]
            v1 = ecs[0]
            for ch in range(1, n_ch):
                v1 = jnp.maximum(v1, ecs[ch])
            v1 = _bmax(v1, lane)
            i1 = jnp.full((_L,), n_e, jnp.int32)
            for ch in range(n_ch):
                idc = lane + ch * _L
                i1 = jnp.minimum(i1, jnp.where(ecs[ch] >= v1, idc, n_e))
            i1 = _bmin(i1, lane)
            e2s = []
            v2 = jnp.full((_L,), -1.0, jnp.float32)
            for ch in range(n_ch):
                idc = lane + ch * _L
                e2 = jnp.where(idc == i1, -1.0, ecs[ch])
                e2s.append(e2)
                v2 = jnp.maximum(v2, e2)
            v2 = _bmax(v2, lane)
            i2 = jnp.full((_L,), n_e, jnp.int32)
            for ch in range(n_ch):
                idc = lane + ch * _L
                i2 = jnp.minimum(i2, jnp.where(e2s[ch] >= v2, idc, n_e))
            i2 = _bmin(i2, lane)
            den = v1 + v2
            wa = v1 / den
            wb = v2 / den
            for ch in range(n_ch):
                idc = lane + ch * _L
                hit1 = jnp.where(idc == i1, 1.0, 0.0)
                hit2 = jnp.where(idc == i2, 1.0, 0.0)
                ct_v[tk, pl.ds(ch * _L, _L)] = hit1 * wa + hit2 * wb
                accs[ch] = accs[ch] + hit1 + hit2
        pltpu.sync_copy(ct_v, ct_ref.at[pl.ds(base, tok_per)])
        for ch in range(n_ch):
            cnt_loc[0, pl.ds(ch * _L, _L)] = accs[ch]
        pltpu.sync_copy(cnt_loc, cnt_hbm.at[pl.ds(s, 1)])

    plsc.subcore_barrier()

    @pl.when(jnp.logical_and(c == 0, s == 0))
    def _compact():
        pltpu.sync_copy(cnt_hbm, cnt_all)
        carry = jnp.zeros((_L,), jnp.float32)
        last = jnp.full((_L,), -1.0, jnp.float32)
        poss = []
        for ch in range(n_ch):
            cnt = cnt_all[0, pl.ds(ch * _L, _L)]
            for sub in range(1, 16):
                cnt = cnt + cnt_all[sub, pl.ds(ch * _L, _L)]
            act = jnp.where(cnt > 0.0, 1.0, 0.0)
            idc = lane + ch * _L
            pos = _prefix(act, lane) + carry - 1.0
            poss.append(jnp.where(cnt > 0.0, pos, -1.0))
            carry = carry + _bsum(act, lane)
            fid = idc.astype(jnp.float32)
            last = jnp.maximum(last, _bmax(fid * act - (1.0 - act), lane))
        splats = []
        for ch in range(n_ch):
            for j in range(_L):
                splats.append((float(ch * _L + j),
                               _shuf(poss[ch], jnp.full((_L,), j, jnp.int32))))
        n_i = carry.astype(jnp.int32)
        last_i = last.astype(jnp.int32)
        for sc in range(n_ch):
            sl = (lane + sc * _L).astype(jnp.float32)
            acc = jnp.zeros((_L,), jnp.float32)
            for e_const, p in splats:
                acc = acc + jnp.where(p == sl, e_const, 0.0)
            idc = lane + sc * _L
            meta_v[pl.ds(sc * _L, _L)] = jnp.where(
                idc < n_i, acc.astype(jnp.int32), last_i)
            meta_v[pl.ds(n_e + sc * _L, _L)] = n_i
        pltpu.sync_copy(meta_v, meta_ref)


def _moe_body(meta_ref, x_ref, ct_ref, w1_ref, b1_ref, w2_ref, b2_ref, o_ref):
    i = pl.program_id(0)
    n_e = ct_ref.shape[1]

    @pl.when(i == 0)
    def _init():
        o_ref[...] = jnp.zeros_like(o_ref)

    @pl.when(i < meta_ref[n_e])
    def _compute():
        x = x_ref[...]
        h = jnp.dot(x, w1_ref[0], preferred_element_type=jnp.float32)
        h = h + b1_ref[0]
        a = jax.nn.gelu(h)
        y = jnp.dot(a, w2_ref[0], preferred_element_type=jnp.float32)
        y = y + b2_ref[0]
        e = meta_ref[i]
        sel = (jax.lax.broadcasted_iota(jnp.int32, (n_e, 1), 0) == e).astype(jnp.float32)
        colw = jnp.dot(ct_ref[...], sel, preferred_element_type=jnp.float32)
        o_ref[...] += colw * y


def kernel(hidden_states, router_logits, w1, b1, w2, b2):
    t, d = hidden_states.shape
    n_e = router_logits.shape[1]
    ffn = w1.shape[2]
    tok_per = t // 16

    mesh = plsc.VectorSubcoreMesh(core_axis_name="c", subcore_axis_name="s")
    router = functools.partial(
        pl.kernel,
        mesh=mesh,
        out_type=[
            jax.ShapeDtypeStruct((t, n_e), jnp.float32),
            jax.ShapeDtypeStruct((2 * n_e,), jnp.int32),
            jax.ShapeDtypeStruct((16, n_e), jnp.float32),
        ],
        scratch_types=[
            pltpu.VMEM((tok_per, n_e), jnp.float32),
            pltpu.VMEM((tok_per, n_e), jnp.float32),
            pltpu.VMEM((1, n_e), jnp.float32),
            pltpu.VMEM((16, n_e), jnp.float32),
            pltpu.VMEM((2 * n_e,), jnp.int32),
        ],
    )(_sc_router_body)
    ct_tm, meta, _ = router(router_logits)

    b1_3 = b1[:, None, :]
    b2_3 = b2[:, None, :]

    grid_spec = pltpu.PrefetchScalarGridSpec(
        num_scalar_prefetch=1,
        grid=(n_e,),
        in_specs=[
            pl.BlockSpec((t, d), lambda i, m: (0, 0)),
            pl.BlockSpec((t, n_e), lambda i, m: (0, 0)),
            pl.BlockSpec((1, d, ffn), lambda i, m: (m[i], 0, 0)),
            pl.BlockSpec((1, 1, ffn), lambda i, m: (m[i], 0, 0)),
            pl.BlockSpec((1, ffn, d), lambda i, m: (m[i], 0, 0)),
            pl.BlockSpec((1, 1, d), lambda i, m: (m[i], 0, 0)),
        ],
        out_specs=pl.BlockSpec((t, d), lambda i, m: (0, 0)),
    )

    out = pl.pallas_call(
        _moe_body,
        grid_spec=grid_spec,
        out_shape=jax.ShapeDtypeStruct((t, d), jnp.float32),
        compiler_params=pltpu.CompilerParams(
            dimension_semantics=("arbitrary",),
        ),
    )(meta, hidden_states, ct_tm, w1, b1_3, w2, b2_3)
    return out


# speedup vs baseline: 1.0570x; 1.0570x over previous
import jax
import jax.numpy as jnp
from jax.experimental import pallas as pl
from jax.experimental.pallas import tpu as pltpu


def _router_body(logits_ref, ct_ref, meta_ref):
    logits = logits_ref[...]
    t, e = logits.shape
    m = jnp.max(logits, axis=-1, keepdims=True)
    ex = jnp.exp(logits - m)
    probs = ex / jnp.sum(ex, axis=-1, keepdims=True)
    col = jax.lax.broadcasted_iota(jnp.int32, (t, e), 1)
    v1 = jnp.max(probs, axis=-1)
    i1 = jnp.min(jnp.where(probs >= v1[:, None], col, e), axis=-1)
    masked = jnp.where(col == i1[:, None], -jnp.inf, probs)
    v2 = jnp.max(masked, axis=-1)
    i2 = jnp.min(jnp.where(masked >= v2[:, None], col, e), axis=-1)
    s = v1 + v2
    wa = (v1 / s)[:, None]
    wb = (v2 / s)[:, None]
    comb = jnp.where(col == i1[:, None], wa, 0.0) + jnp.where(col == i2[:, None], wb, 0.0)
    ct_ref[...] = comb.T

    actf = (jnp.max(comb, axis=0, keepdims=True) > 0.0).astype(jnp.float32)
    r2 = jax.lax.broadcasted_iota(jnp.int32, (e, e), 0)
    c2 = jax.lax.broadcasted_iota(jnp.int32, (e, e), 1)
    tri = (r2 <= c2).astype(jnp.float32)
    cums = jnp.dot(actf, tri, preferred_element_type=jnp.float32)
    n = cums[0, e - 1]
    pos_t = (cums - 1.0).T
    match = (pos_t == c2.astype(jnp.float32)) & (actf.T > 0.0)
    erow = jax.lax.broadcasted_iota(jnp.int32, (1, e), 1).astype(jnp.float32)
    ids_sorted = jnp.dot(erow, match.astype(jnp.float32), preferred_element_type=jnp.float32)
    last = jnp.max(erow * actf - (1.0 - actf))
    ids_final = jnp.where(erow < n, ids_sorted, last)
    meta = jnp.concatenate([ids_final, jnp.full((1, e), n)], axis=1)
    meta_ref[...] = meta.astype(jnp.int32)


def _moe_body(meta_ref, x_ref, ct_ref, w1_ref, b1_ref, w2_ref, b2_ref, o_ref):
    i = pl.program_id(0)
    n_e = ct_ref.shape[0]

    @pl.when(i == 0)
    def _init():
        o_ref[...] = jnp.zeros_like(o_ref)

    @pl.when(i < meta_ref[n_e])
    def _compute():
        x = x_ref[...]
        h = jnp.dot(x, w1_ref[0], preferred_element_type=jnp.float32)
        h = h + b1_ref[0]
        a = jax.nn.gelu(h)
        y = jnp.dot(a, w2_ref[0], preferred_element_type=jnp.float32)
        y = y + b2_ref[0]
        e = meta_ref[i]
        colw = ct_ref[e, :]
        o_ref[...] += colw[:, None] * y


def kernel(hidden_states, router_logits, w1, b1, w2, b2):
    t, d = hidden_states.shape
    n_e = router_logits.shape[1]
    ffn = w1.shape[2]

    ct, meta = pl.pallas_call(
        _router_body,
        out_shape=[
            jax.ShapeDtypeStruct((n_e, t), jnp.float32),
            jax.ShapeDtypeStruct((1, 2 * n_e), jnp.int32),
        ],
    )(router_logits)
    meta = meta.reshape((2 * n_e,))

    b1_3 = b1[:, None, :]
    b2_3 = b2[:, None, :]

    grid_spec = pltpu.PrefetchScalarGridSpec(
        num_scalar_prefetch=1,
        grid=(n_e,),
        in_specs=[
            pl.BlockSpec((t, d), lambda i, m: (0, 0)),
            pl.BlockSpec((n_e, t), lambda i, m: (0, 0)),
            pl.BlockSpec((1, d, ffn), lambda i, m: (m[i], 0, 0)),
            pl.BlockSpec((1, 1, ffn), lambda i, m: (m[i], 0, 0)),
            pl.BlockSpec((1, ffn, d), lambda i, m: (m[i], 0, 0)),
            pl.BlockSpec((1, 1, d), lambda i, m: (m[i], 0, 0)),
        ],
        out_specs=pl.BlockSpec((t, d), lambda i, m: (0, 0)),
    )

    out = pl.pallas_call(
        _moe_body,
        grid_spec=grid_spec,
        out_shape=jax.ShapeDtypeStruct((t, d), jnp.float32),
        compiler_params=pltpu.CompilerParams(
            dimension_semantics=("arbitrary",),
        ),
    )(meta, hidden_states, ct, w1, b1_3, w2, b2_3)
    return out
